# baseline (speedup 1.0000x reference)
import functools

import jax
import jax.numpy as jnp
from jax import lax
from jax.experimental import pallas as pl
from jax.experimental.pallas import tpu as pltpu
from jax.experimental.pallas import tpu_sc as plsc

_B = 128
_V = 100000
_A_PAD = 112
_NC = 2
_NS = 16
_ROWS = _B // (_NC * _NS)
_L = 16
_SEG = 20000
_NSEG = _V // _SEG


def _body(scores_hbm, allowed_hbm, out_hbm, seg_v, allowed_v, idx_v, vals_v,
          gsem, fsem):
    wid = lax.axis_index("s") * _NC + lax.axis_index("c")

    neg = jnp.full((_L,), -jnp.inf, jnp.float32)

    @pl.loop(0, _SEG // _L)
    def _fill(i):
        seg_v[pl.ds(i * _L, _L)] = neg

    pltpu.sync_copy(allowed_hbm, allowed_v)
    for r in range(_ROWS):
        base = (wid * _ROWS + r) * _V
        for c in range(_A_PAD // _L):
            a16 = allowed_v[pl.ds(_L * c, _L)]
            idx_v[pl.ds(r * _A_PAD + _L * c, _L)] = a16 + base
    gather = pltpu.make_async_copy(scores_hbm.at[idx_v], vals_v, gsem)
    gather.start()

    fills = [
        pltpu.make_async_copy(
            seg_v,
            out_hbm.at[pl.ds((wid * _ROWS + r) * _V + s * _SEG, _SEG)],
            fsem)
        for r in range(_ROWS) for s in range(_NSEG)
    ]
    for f in fills:
        f.start()
    gather.wait()
    for f in fills:
        f.wait()
    pltpu.async_copy(vals_v, out_hbm.at[idx_v], gsem).wait()


_sc_mask = functools.partial(
    pl.kernel,
    out_type=jax.ShapeDtypeStruct((_B * _V,), jnp.float32),
    mesh=plsc.VectorSubcoreMesh(
        core_axis_name="c", subcore_axis_name="s",
        num_cores=_NC, num_subcores=_NS),
    scratch_types=[
        pltpu.VMEM((_SEG,), jnp.float32),
        pltpu.VMEM((_A_PAD,), jnp.int32),
        pltpu.VMEM((_ROWS * _A_PAD,), jnp.int32),
        pltpu.VMEM((_ROWS * _A_PAD,), jnp.float32),
        pltpu.SemaphoreType.DMA,
        pltpu.SemaphoreType.DMA,
    ],
)(_body)


def kernel(input_ids, scores, allowed_token_ids):
    del input_ids
    a = allowed_token_ids.astype(jnp.int32)
    a_pad = jnp.concatenate(
        [a, jnp.broadcast_to(a[-1:], (_A_PAD - a.shape[0],))])
    flat = _sc_mask(scores.reshape(-1), a_pad)
    return flat.reshape(_B, _V)


# speedup vs baseline: 2.2951x; 2.2951x over previous
import jax
import jax.numpy as jnp
from jax.experimental import pallas as pl
from jax.experimental.pallas import tpu as pltpu

_B = 128
_V = 100000
_A = 100
_WIN = 128


def _patch_body(ws_ref, canvas_ref, scores_ref, allowed_ref, out_ref,
                wbuf, pbuf, lsems, ssems):
    del canvas_ref
    loads = []
    for k in range(_A):
        ws = pl.multiple_of(ws_ref[k], _WIN)
        c = pltpu.make_async_copy(
            scores_ref.at[:, pl.ds(ws, _WIN)], wbuf.at[k], lsems.at[k])
        c.start()
        loads.append((ws, c))
    a = allowed_ref[...]
    stores = []
    for k in range(_A):
        ws, c = loads[k]
        c.wait()
        cols = ws + jax.lax.broadcasted_iota(jnp.int32, (1, _WIN), 1)
        hit = jnp.any(a == cols, axis=0, keepdims=True)
        pbuf[k] = jnp.where(hit, wbuf[k], -jnp.inf)
        s = pltpu.make_async_copy(
            pbuf.at[k], out_ref.at[:, pl.ds(ws, _WIN)], ssems.at[k])
        s.start()
        stores.append(s)
    for s in stores:
        s.wait()


def kernel(input_ids, scores, allowed_token_ids):
    B, V = scores.shape
    a = allowed_token_ids.astype(jnp.int32)
    a_pad = jnp.pad(a, (0, 128 - _A), constant_values=-1).reshape(128, 1)
    ws = (a // _WIN) * _WIN
    zero = (input_ids[0, 0] * 0).astype(jnp.float32)
    canvas = jnp.full((B, V), -jnp.inf, jnp.float32) + zero
    return pl.pallas_call(
        _patch_body,
        in_specs=[
            pl.BlockSpec(memory_space=pltpu.MemorySpace.SMEM),
            pl.BlockSpec(memory_space=pltpu.MemorySpace.HBM),
            pl.BlockSpec(memory_space=pltpu.MemorySpace.HBM),
            pl.BlockSpec((128, 1), lambda: (0, 0)),
        ],
        out_specs=pl.BlockSpec(memory_space=pltpu.MemorySpace.HBM),
        out_shape=jax.ShapeDtypeStruct((B, V), jnp.float32),
        input_output_aliases={1: 0},
        scratch_shapes=[
            pltpu.VMEM((_A, 128, _WIN), jnp.float32),
            pltpu.VMEM((_A, 128, _WIN), jnp.float32),
            pltpu.SemaphoreType.DMA((_A,)),
            pltpu.SemaphoreType.DMA((_A,)),
        ],
    )(ws, canvas, scores, a_pad)


# speedup vs baseline: 2.4295x; 1.0586x over previous
import jax
import jax.numpy as jnp
from jax.experimental import pallas as pl
from jax.experimental.pallas import tpu as pltpu

_B = 128
_V = 100000
_A = 100
_WIN = 128
_FW = 2048
_NF = 48
_TAIL = _V - _NF * _FW


def _body(ws_ref, scores_ref, allowed_ref, out_ref,
          const_v, tail_v, wbuf, pbuf, fsems, lsems, ssems):
    loads = []
    for k in range(_A):
        ws = pl.multiple_of(ws_ref[k], _WIN)
        c = pltpu.make_async_copy(
            scores_ref.at[:, pl.ds(ws, _WIN)], wbuf.at[k], lsems.at[k])
        c.start()
        loads.append((ws, c))

    const_v[...] = jnp.full_like(const_v, -jnp.inf)
    tail_v[...] = jnp.full_like(tail_v, -jnp.inf)
    fills = [
        pltpu.make_async_copy(const_v, out_ref.at[:, pl.ds(t * _FW, _FW)],
                              fsems.at[t % 8])
        for t in range(_NF)
    ]
    fills.append(pltpu.make_async_copy(
        tail_v, out_ref.at[:, pl.ds(_NF * _FW, _TAIL)], fsems.at[_NF % 8]))
    for f in fills:
        f.start()

    a = allowed_ref[...]
    for k in range(_A):
        ws, c = loads[k]
        c.wait()
        cols = ws + jax.lax.broadcasted_iota(jnp.int32, (1, _WIN), 1)
        hit = jnp.any(a == cols, axis=0, keepdims=True)
        pbuf[k] = jnp.where(hit, wbuf[k], -jnp.inf)

    for f in fills:
        f.wait()
    stores = []
    for k in range(_A):
        ws, _ = loads[k]
        s = pltpu.make_async_copy(
            pbuf.at[k], out_ref.at[:, pl.ds(ws, _WIN)], ssems.at[k])
        s.start()
        stores.append(s)
    for s in stores:
        s.wait()


def kernel(input_ids, scores, allowed_token_ids):
    del input_ids
    B, V = scores.shape
    a = allowed_token_ids.astype(jnp.int32)
    a_pad = jnp.pad(a, (0, 128 - _A), constant_values=-1).reshape(128, 1)
    ws = (a // _WIN) * _WIN
    return pl.pallas_call(
        _body,
        in_specs=[
            pl.BlockSpec(memory_space=pltpu.MemorySpace.SMEM),
            pl.BlockSpec(memory_space=pltpu.MemorySpace.HBM),
            pl.BlockSpec((128, 1), lambda: (0, 0)),
        ],
        out_specs=pl.BlockSpec(memory_space=pltpu.MemorySpace.HBM),
        out_shape=jax.ShapeDtypeStruct((B, V), jnp.float32),
        scratch_shapes=[
            pltpu.VMEM((128, _FW), jnp.float32),
            pltpu.VMEM((128, _TAIL), jnp.float32),
            pltpu.VMEM((_A, 128, _WIN), jnp.float32),
            pltpu.VMEM((_A, 128, _WIN), jnp.float32),
            pltpu.SemaphoreType.DMA((8,)),
            pltpu.SemaphoreType.DMA((_A,)),
            pltpu.SemaphoreType.DMA((_A,)),
        ],
    )(ws, scores, a_pad)
